# Initial kernel scaffold; baseline (speedup 1.0000x reference)
import jax
import jax.numpy as jnp
from jax.experimental import pallas as pl


def kernel(a_hat, x, w1, b1, w2, b2):
    raise NotImplementedError("write your pallas kernel here")


# speedup vs baseline: 1.1178x; 1.1178x over previous
import functools

import jax
import jax.numpy as jnp
from jax.experimental import pallas as pl
from jax.experimental.pallas import tpu as pltpu

_VMEM = 100 * 1024 * 1024


def _xw_kernel(x_ref, w1_ref, o_ref):
    o_ref[...] = jnp.dot(x_ref[...].astype(jnp.bfloat16), w1_ref[...],
                         preferred_element_type=jnp.float32
                         ).astype(o_ref.dtype)


def _layer1_kernel(a_ref, xw_ref, b1_ref, w2_ref, o_ref):
    acc = jnp.dot(a_ref[...], xw_ref[...],
                  preferred_element_type=jnp.float32)
    h = jnp.maximum(acc + b1_ref[...], 0.0)
    o_ref[...] = jnp.dot(h.astype(jnp.bfloat16), w2_ref[...],
                         preferred_element_type=jnp.float32
                         ).astype(o_ref.dtype)


def _layer2_kernel(a_ref, hw_ref, b2_ref, o_ref, *, n_classes):
    acc = jnp.dot(a_ref[...], hw_ref[...],
                  preferred_element_type=jnp.float32)
    z = acc + b2_ref[...]
    col = jax.lax.broadcasted_iota(jnp.int32, z.shape, dimension=1)
    valid = col < n_classes
    z = jnp.where(valid, z, -jnp.inf)
    mx = jnp.max(z, axis=1, keepdims=True)
    s = z - mx
    lse = jnp.log(jnp.sum(jnp.exp(s), axis=1, keepdims=True))
    o_ref[...] = jnp.where(valid, s - lse, 0.0)


def kernel(a_hat, x, w1, b1, w2, b2):
    n, f = x.shape
    hidden = w1.shape[1]
    n_classes = w2.shape[1]
    cp = 8

    a_p = a_hat
    if a_p.shape != (n, n) or a_p.dtype != jnp.bfloat16:
        a_p = jnp.zeros((n, n), jnp.bfloat16).at[:n, :n].set(
            a_hat[:n, :n].astype(jnp.bfloat16))

    w1_b = w1.astype(jnp.bfloat16)
    b1_r = b1.astype(jnp.float32).reshape(1, hidden)
    w2_p = jnp.zeros((hidden, cp), jnp.bfloat16).at[:, :n_classes].set(
        w2.astype(jnp.bfloat16))
    b2_p = jnp.zeros((1, cp), jnp.float32).at[0, :n_classes].set(
        b2.astype(jnp.float32))

    tile = min(512, n)
    grid = (n // tile,)

    xw = pl.pallas_call(
        _xw_kernel,
        out_shape=jax.ShapeDtypeStruct((n, hidden), jnp.bfloat16),
        grid=grid,
        in_specs=[pl.BlockSpec((tile, f), lambda i: (i, 0)),
                  pl.BlockSpec((f, hidden), lambda i: (0, 0))],
        out_specs=pl.BlockSpec((tile, hidden), lambda i: (i, 0)),
        compiler_params=pltpu.CompilerParams(
            dimension_semantics=("parallel",),
            vmem_limit_bytes=_VMEM,
        ),
    )(x, w1_b)

    hw = pl.pallas_call(
        _layer1_kernel,
        out_shape=jax.ShapeDtypeStruct((n, cp), jnp.bfloat16),
        grid=grid,
        in_specs=[pl.BlockSpec((tile, n), lambda i: (i, 0)),
                  pl.BlockSpec((n, hidden), lambda i: (0, 0)),
                  pl.BlockSpec((1, hidden), lambda i: (0, 0)),
                  pl.BlockSpec((hidden, cp), lambda i: (0, 0))],
        out_specs=pl.BlockSpec((tile, cp), lambda i: (i, 0)),
        compiler_params=pltpu.CompilerParams(
            dimension_semantics=("parallel",),
            vmem_limit_bytes=_VMEM,
        ),
    )(a_p, xw, b1_r, w2_p)

    out_p = pl.pallas_call(
        functools.partial(_layer2_kernel, n_classes=n_classes),
        out_shape=jax.ShapeDtypeStruct((n, cp), jnp.float32),
        grid=grid,
        in_specs=[pl.BlockSpec((tile, n), lambda i: (i, 0)),
                  pl.BlockSpec((n, cp), lambda i: (0, 0)),
                  pl.BlockSpec((1, cp), lambda i: (0, 0))],
        out_specs=pl.BlockSpec((tile, cp), lambda i: (i, 0)),
        compiler_params=pltpu.CompilerParams(
            dimension_semantics=("parallel",),
            vmem_limit_bytes=_VMEM,
        ),
    )(a_p, hw, b2_p)

    return out_p[:, :n_classes]


# speedup vs baseline: 1.2884x; 1.1526x over previous
import functools

import jax
import jax.numpy as jnp
from jax.experimental import pallas as pl
from jax.experimental.pallas import tpu as pltpu

_VMEM = 100 * 1024 * 1024


def _xw_kernel(x_ref, w1_ref, o_ref):
    o_ref[...] = jnp.dot(x_ref[...].astype(jnp.bfloat16), w1_ref[...],
                         preferred_element_type=jnp.float32
                         ).astype(o_ref.dtype)


def _fused_kernel(a_hbm, xw_ref, b1_ref, w2_ref, b2_ref, out_ref,
                  a_vmem, hw_ref, sem, *, n_classes, tile, nt):
    p = pl.program_id(0)
    i = pl.program_id(1)

    @pl.when((p == 0) & (i == 0))
    def _():
        for s in range(nt):
            pltpu.make_async_copy(
                a_hbm.at[pl.ds(s * tile, tile), :],
                a_vmem.at[pl.ds(s * tile, tile), :],
                sem.at[s],
            ).start()

    @pl.when(p == 0)
    def _():
        pltpu.make_async_copy(
            a_hbm.at[pl.ds(i * tile, tile), :],
            a_vmem.at[pl.ds(i * tile, tile), :],
            sem.at[i],
        ).wait()
        a_blk = a_vmem[pl.ds(pl.multiple_of(i * tile, tile), tile), :]
        acc = jnp.dot(a_blk, xw_ref[...], preferred_element_type=jnp.float32)
        h = jnp.maximum(acc + b1_ref[...], 0.0)
        hw_ref[pl.ds(pl.multiple_of(i * tile, tile), tile), :] = jnp.dot(
            h.astype(jnp.bfloat16), w2_ref[...],
            preferred_element_type=jnp.float32).astype(jnp.bfloat16)

    @pl.when(p == 1)
    def _():
        a_blk = a_vmem[pl.ds(pl.multiple_of(i * tile, tile), tile), :]
        acc = jnp.dot(a_blk, hw_ref[...], preferred_element_type=jnp.float32)
        z = acc + b2_ref[...]
        col = jax.lax.broadcasted_iota(jnp.int32, z.shape, 1)
        valid = col < n_classes
        z = jnp.where(valid, z, -jnp.inf)
        mx = jnp.max(z, axis=1, keepdims=True)
        s = z - mx
        lse = jnp.log(jnp.sum(jnp.exp(s), axis=1, keepdims=True))
        out_ref[pl.ds(pl.multiple_of(i * tile, tile), tile), :] = \
            jnp.where(valid, s - lse, 0.0)


def kernel(a_hat, x, w1, b1, w2, b2):
    n, f = x.shape
    hidden = w1.shape[1]
    n_classes = w2.shape[1]
    cp = 8

    a_p = a_hat
    if a_p.shape != (n, n) or a_p.dtype != jnp.bfloat16:
        a_p = jnp.zeros((n, n), jnp.bfloat16).at[:n, :n].set(
            a_hat[:n, :n].astype(jnp.bfloat16))

    w1_b = w1.astype(jnp.bfloat16)
    b1_r = b1.astype(jnp.float32).reshape(1, hidden)
    w2_p = jnp.zeros((hidden, cp), jnp.bfloat16).at[:, :n_classes].set(
        w2.astype(jnp.bfloat16))
    b2_p = jnp.zeros((1, cp), jnp.float32).at[0, :n_classes].set(
        b2.astype(jnp.float32))

    tile = min(512, n)
    nt = n // tile

    xw = pl.pallas_call(
        _xw_kernel,
        out_shape=jax.ShapeDtypeStruct((n, hidden), jnp.bfloat16),
        grid=(nt,),
        in_specs=[pl.BlockSpec((tile, f), lambda i: (i, 0)),
                  pl.BlockSpec((f, hidden), lambda i: (0, 0))],
        out_specs=pl.BlockSpec((tile, hidden), lambda i: (i, 0)),
        compiler_params=pltpu.CompilerParams(
            dimension_semantics=("parallel",),
            vmem_limit_bytes=_VMEM,
        ),
    )(x, w1_b)

    out_p = pl.pallas_call(
        functools.partial(_fused_kernel, n_classes=n_classes, tile=tile,
                          nt=nt),
        out_shape=jax.ShapeDtypeStruct((n, cp), jnp.float32),
        grid=(2, nt),
        in_specs=[pl.BlockSpec(memory_space=pl.ANY),
                  pl.BlockSpec((n, hidden), lambda p, i: (0, 0)),
                  pl.BlockSpec((1, hidden), lambda p, i: (0, 0)),
                  pl.BlockSpec((hidden, cp), lambda p, i: (0, 0)),
                  pl.BlockSpec((1, cp), lambda p, i: (0, 0))],
        out_specs=pl.BlockSpec((n, cp), lambda p, i: (0, 0)),
        scratch_shapes=[
            pltpu.VMEM((n, n), jnp.bfloat16),
            pltpu.VMEM((n, cp), jnp.bfloat16),
            pltpu.SemaphoreType.DMA((nt,)),
        ],
        compiler_params=pltpu.CompilerParams(
            dimension_semantics=("arbitrary", "arbitrary"),
            vmem_limit_bytes=_VMEM,
        ),
    )(a_p, xw, b1_r, w2_p, b2_p)

    return out_p[:, :n_classes]
